# baseline (speedup 1.0000x reference)
import functools

import jax
import jax.numpy as jnp
from jax.experimental import pallas as pl
from jax.experimental.pallas import tpu as pltpu

T, D, F, E, K = 2048, 1024, 2048, 8, 2
TB = 256
NT = T // TB


def _moe_body(x_ref, wr_ref, wg_ref, wu_ref, wd_ref, y_ref, g_ref):
    e = pl.program_id(0)
    t = pl.program_id(1)
    rows = pl.ds(t * TB, TB)
    xb = x_ref[rows, :]

    @pl.when(e == 0)
    def _gating():
        scores = jax.lax.dot_general(
            xb, wr_ref[...], (((1,), (1,)), ((), ())),
            preferred_element_type=jnp.float32)
        idx = jax.lax.broadcasted_iota(jnp.int32, (TB, E), 1)
        m1 = jnp.max(scores, axis=1, keepdims=True)
        i1 = jnp.min(jnp.where(scores == m1, idx, E), axis=1, keepdims=True)
        s2 = jnp.where(idx == i1, -jnp.inf, scores)
        m2 = jnp.max(s2, axis=1, keepdims=True)
        i2 = jnp.min(jnp.where(s2 == m2, idx, E), axis=1, keepdims=True)
        q = jnp.exp(m2 - m1)
        p1 = 1.0 / (1.0 + q)
        p2 = q * p1
        gating = jnp.where(idx == i1, p1, 0.0) + jnp.where(idx == i2, p2, 0.0)
        g_ref[rows, :] = gating

    xb16 = xb.astype(jnp.bfloat16)
    wg = wg_ref[0]
    wu = wu_ref[0]
    wd = wd_ref[0]
    h1 = jax.lax.dot_general(xb16, wg, (((1,), (1,)), ((), ())),
                             preferred_element_type=jnp.float32)
    h2 = jax.lax.dot_general(xb16, wu, (((1,), (1,)), ((), ())),
                             preferred_element_type=jnp.float32)
    h = (h1 / (1.0 + jnp.exp(-h1))) * h2
    out = jax.lax.dot_general(h.astype(jnp.bfloat16), wd,
                              (((1,), (1,)), ((), ())),
                              preferred_element_type=jnp.float32)
    gb = g_ref[rows, :]
    eidx = jax.lax.broadcasted_iota(jnp.int32, (TB, E), 1)
    g_col = jnp.sum(jnp.where(eidx == e, gb, 0.0), axis=1, keepdims=True)
    contrib = g_col * out

    @pl.when(e == 0)
    def _init():
        y_ref[rows, :] = contrib

    @pl.when(e > 0)
    def _acc():
        y_ref[rows, :] += contrib


@jax.jit
def kernel(x, W_router, W_gate, W_up, W_down):
    wg16 = W_gate.astype(jnp.bfloat16)
    wu16 = W_up.astype(jnp.bfloat16)
    wd16 = W_down.astype(jnp.bfloat16)
    y = pl.pallas_call(
        _moe_body,
        grid=(E, NT),
        in_specs=[
            pl.BlockSpec((T, D), lambda e, t: (0, 0)),
            pl.BlockSpec((E, D), lambda e, t: (0, 0)),
            pl.BlockSpec((1, F, D), lambda e, t: (e, 0, 0)),
            pl.BlockSpec((1, F, D), lambda e, t: (e, 0, 0)),
            pl.BlockSpec((1, D, F), lambda e, t: (e, 0, 0)),
        ],
        out_specs=pl.BlockSpec((T, D), lambda e, t: (0, 0)),
        out_shape=jax.ShapeDtypeStruct((T, D), jnp.float32),
        scratch_shapes=[pltpu.VMEM((T, E), jnp.float32)],
        compiler_params=pltpu.CompilerParams(
            dimension_semantics=("arbitrary", "arbitrary"),
        ),
    )(x, W_router, wg16, wu16, wd16)
    return y


# speedup vs baseline: 1.0545x; 1.0545x over previous
import functools

import jax
import jax.numpy as jnp
from jax import lax
from jax.experimental import pallas as pl
from jax.experimental.pallas import tpu as pltpu
from jax.experimental.pallas import tpu_sc as plsc

T, D, F, E, K = 2048, 1024, 2048, 8, 2
P = K * T
TB = 128
NBMAX = 40
NS = NBMAX * TB
BE_PAD = 64

NCORES, NSUB = 2, 16
NW = NCORES * NSUB
PPT = P // NW
DCH = 32
TPT = T // NW
CCH = 16


def _prefix_lanes(a, n):
    sh = 1
    while sh < n:
        a = a + jnp.pad(a, ((0, 0), (sh, 0)))[:, :n]
        sh *= 2
    return a


def _meta_body(x_ref, wr_ref, pos_ref, pp_ref, be_ref, nb_ref):
    x = x_ref[...]
    wr = wr_ref[...]
    scores = lax.dot_general(wr, x, (((1,), (1,)), ((), ())),
                             preferred_element_type=jnp.float32)
    eidx = lax.broadcasted_iota(jnp.int32, (E, T), 0)
    m1 = jnp.max(scores, axis=0, keepdims=True)
    i1 = jnp.min(jnp.where(scores == m1, eidx, E), axis=0, keepdims=True)
    s2 = jnp.where(eidx == i1, -jnp.inf, scores)
    m2 = jnp.max(s2, axis=0, keepdims=True)
    i2 = jnp.min(jnp.where(s2 == m2, eidx, E), axis=0, keepdims=True)
    q = jnp.exp(m2 - m1)
    p1 = 1.0 / (1.0 + q)
    p2 = q * p1

    e_pair = jnp.concatenate([i1, i2], axis=1)
    p_pair = jnp.concatenate([p1, p2], axis=1)
    eiota = lax.broadcasted_iota(jnp.int32, (E, P), 0)
    onehot = (e_pair == eiota).astype(jnp.int32)

    csum = _prefix_lanes(onehot, P)
    rank = jnp.sum(onehot * csum, axis=0, keepdims=True) - 1
    count = csum[:, P - 1:P]
    nbk = (count + TB - 1) // TB
    incl = nbk
    for sh in (1, 2, 4):
        incl = incl + jnp.pad(incl, ((sh, 0), (0, 0)))[:E, :]
    bo = incl - nbk
    poff = jnp.sum(onehot * (bo * TB), axis=0, keepdims=True)
    pos = poff + rank
    total = jnp.max(incl)

    biota = lax.broadcasted_iota(jnp.int32, (E, BE_PAD), 1)
    be = jnp.sum((biota >= incl).astype(jnp.int32), axis=0, keepdims=True)
    e8 = lax.broadcasted_iota(jnp.int32, (E, 1), 0)
    be_last = jnp.max(jnp.where(nbk > 0, e8, 0))
    bvec = lax.broadcasted_iota(jnp.int32, (1, BE_PAD), 1)
    be = jnp.where(bvec >= total, be_last, jnp.minimum(be, E - 1))

    pos_ref[...] = pos
    pp_ref[...] = p_pair
    be_ref[...] = be
    nb_ref[...] = jnp.full((1, 8), total, jnp.int32)


_meta_call = pl.pallas_call(
    _meta_body,
    in_specs=[pl.BlockSpec((T, D), lambda: (0, 0)),
              pl.BlockSpec((E, D), lambda: (0, 0))],
    out_specs=[pl.BlockSpec((1, P), lambda: (0, 0)),
               pl.BlockSpec((1, P), lambda: (0, 0)),
               pl.BlockSpec((1, BE_PAD), lambda: (0, 0)),
               pl.BlockSpec((1, 8), lambda: (0, 0))],
    out_shape=[jax.ShapeDtypeStruct((1, P), jnp.int32),
               jax.ShapeDtypeStruct((1, P), jnp.float32),
               jax.ShapeDtypeStruct((1, BE_PAD), jnp.int32),
               jax.ShapeDtypeStruct((1, 8), jnp.int32)],
)


def _dispatch_body(x_hbm, pos_hbm, pp_hbm, xs_hbm, ws_hbm,
                   rows_v, idx_v, posall_v, pall_v, wsort_v):
    wid = lax.axis_index("s") * NCORES + lax.axis_index("c")
    base = wid * PPT
    tok0 = base - jnp.where(base >= T, T, 0)
    for c in range(PPT // DCH):
        pltpu.sync_copy(x_hbm.at[pl.ds(tok0 + c * DCH, DCH)], rows_v)
        pltpu.sync_copy(pos_hbm.at[pl.ds(base + c * DCH, DCH)], idx_v)
        pltpu.sync_copy(rows_v, xs_hbm.at[idx_v])

    @pl.when(wid == 0)
    def _weights():
        zeros16 = jnp.zeros((16,), jnp.float32)

        def _zero(i, _):
            wsort_v[pl.ds(i * 16, 16)] = zeros16
            return 0
        lax.fori_loop(0, NS // 16, _zero, 0)
        pltpu.sync_copy(pos_hbm, posall_v)
        pltpu.sync_copy(pp_hbm, pall_v)

        def _scat(c, _):
            k = posall_v[pl.ds(c * 16, 16)]
            w = pall_v[pl.ds(c * 16, 16)]
            plsc.store_scatter(wsort_v, [k], w)
            return 0
        lax.fori_loop(0, P // 16, _scat, 0)
        pltpu.sync_copy(wsort_v, ws_hbm)


_dispatch_call = pl.kernel(
    _dispatch_body,
    out_type=[jax.ShapeDtypeStruct((NS, D), jnp.float32),
              jax.ShapeDtypeStruct((NS,), jnp.float32)],
    mesh=plsc.VectorSubcoreMesh(core_axis_name="c", subcore_axis_name="s",
                                num_cores=NCORES, num_subcores=NSUB),
    scratch_types=[pltpu.VMEM((DCH, D), jnp.float32),
                   pltpu.VMEM((DCH,), jnp.int32),
                   pltpu.VMEM((P,), jnp.int32),
                   pltpu.VMEM((P,), jnp.float32),
                   pltpu.VMEM((NS,), jnp.float32)],
    compiler_params=pltpu.CompilerParams(needs_layout_passes=False),
)


def _mlp_body(be_ref, nb_ref, xs_ref, ws_ref, wg_ref, wu_ref, wd_ref, out_ref):
    b = pl.program_id(0)

    @pl.when(b < nb_ref[0])
    def _():
        xb = xs_ref[...].astype(jnp.bfloat16)
        wg = wg_ref[0]
        wu = wu_ref[0]
        wd = wd_ref[0]
        h1 = lax.dot_general(xb, wg, (((1,), (1,)), ((), ())),
                             preferred_element_type=jnp.float32)
        h2 = lax.dot_general(xb, wu, (((1,), (1,)), ((), ())),
                             preferred_element_type=jnp.float32)
        h = (h1 / (1.0 + jnp.exp(-h1))) * h2
        out = lax.dot_general(h.astype(jnp.bfloat16), wd,
                              (((1,), (1,)), ((), ())),
                              preferred_element_type=jnp.float32)
        w = ws_ref[0]
        ri = lax.broadcasted_iota(jnp.int32, (TB, TB), 0)
        ci = lax.broadcasted_iota(jnp.int32, (TB, TB), 1)
        diag = jnp.where(ri == ci, jnp.broadcast_to(w, (TB, TB)), 0.0)
        out_ref[...] = lax.dot_general(
            diag.astype(jnp.bfloat16), out.astype(jnp.bfloat16),
            (((1,), (0,)), ((), ())), preferred_element_type=jnp.float32)


_mlp_call = pl.pallas_call(
    _mlp_body,
    grid_spec=pltpu.PrefetchScalarGridSpec(
        num_scalar_prefetch=2,
        grid=(NBMAX,),
        in_specs=[
            pl.BlockSpec((TB, D), lambda b, be, nb: (jnp.minimum(b, nb[0] - 1), 0)),
            pl.BlockSpec((1, 1, TB), lambda b, be, nb: (jnp.minimum(b, nb[0] - 1), 0, 0)),
            pl.BlockSpec((1, F, D), lambda b, be, nb: (be[b], 0, 0)),
            pl.BlockSpec((1, F, D), lambda b, be, nb: (be[b], 0, 0)),
            pl.BlockSpec((1, D, F), lambda b, be, nb: (be[b], 0, 0)),
        ],
        out_specs=pl.BlockSpec((TB, D), lambda b, be, nb: (jnp.minimum(b, nb[0] - 1), 0)),
    ),
    out_shape=jax.ShapeDtypeStruct((NS, D), jnp.float32),
    compiler_params=pltpu.CompilerParams(
        dimension_semantics=("arbitrary",),
    ),
)


def _combine_body(os_hbm, pos_hbm, y_hbm, a_v, b_v, ia_v, ib_v, sem_a, sem_b):
    wid = lax.axis_index("s") * NCORES + lax.axis_index("c")
    for c in range(TPT // CCH):
        tb = wid * TPT + c * CCH
        pltpu.sync_copy(pos_hbm.at[pl.ds(tb, CCH)], ia_v)
        pltpu.sync_copy(pos_hbm.at[pl.ds(T + tb, CCH)], ib_v)
        cp_a = pltpu.async_copy(os_hbm.at[ia_v], a_v, sem_a)
        cp_b = pltpu.async_copy(os_hbm.at[ib_v], b_v, sem_b)
        cp_a.wait()
        cp_b.wait()
        for i in range(CCH):
            def _add(s, _):
                sl = pl.ds(s * 16, 16)
                a_v[i, sl] = a_v[i, sl] + b_v[i, sl]
                return 0
            lax.fori_loop(0, D // 16, _add, 0)
        pltpu.sync_copy(a_v, y_hbm.at[pl.ds(tb, CCH)])


_combine_call = pl.kernel(
    _combine_body,
    out_type=jax.ShapeDtypeStruct((T, D), jnp.float32),
    mesh=plsc.VectorSubcoreMesh(core_axis_name="c", subcore_axis_name="s",
                                num_cores=NCORES, num_subcores=NSUB),
    scratch_types=[pltpu.VMEM((CCH, D), jnp.float32),
                   pltpu.VMEM((CCH, D), jnp.float32),
                   pltpu.VMEM((CCH,), jnp.int32),
                   pltpu.VMEM((CCH,), jnp.int32),
                   pltpu.SemaphoreType.DMA,
                   pltpu.SemaphoreType.DMA],
)


@jax.jit
def kernel(x, W_router, W_gate, W_up, W_down):
    pos2, pp2, be2, nb2 = _meta_call(x, W_router)
    pos = pos2.reshape(P)
    pp = pp2.reshape(P)
    be = be2.reshape(BE_PAD)
    nb = nb2.reshape(8)[:1]
    xs, ws = _dispatch_call(x, pos, pp)
    ws3 = ws.reshape(NBMAX, 1, TB)
    wg16 = W_gate.astype(jnp.bfloat16)
    wu16 = W_up.astype(jnp.bfloat16)
    wd16 = W_down.astype(jnp.bfloat16)
    out_sorted = _mlp_call(be, nb, xs, ws3, wg16, wu16, wd16)
    return _combine_call(out_sorted, pos)
